# baseline (speedup 1.0000x reference)
import functools

import jax
import jax.numpy as jnp
from jax import lax
from jax.experimental import pallas as pl
from jax.experimental.pallas import tpu as pltpu
from jax.experimental.pallas import tpu_sc as plsc

K = 8
D = 128
T = 262144

NC = 2
NS = 16
NW = NC * NS
TOK_PER_W = T // NW
CHUNK = 128
NCHUNK = TOK_PER_W // CHUNK

NBUF = 4


def _proj_body(table_ref, w_ref, b_ref, out_ref):
    out_ref[...] = (
        jnp.dot(table_ref[...], w_ref[...], preferred_element_type=jnp.float32)
        + b_ref[...]
    )


def _project_table(table, W, b):
    return pl.pallas_call(
        _proj_body,
        out_shape=jax.ShapeDtypeStruct((K, D), jnp.float32),
    )(table, W, b.reshape(1, D))


_sc_mesh = plsc.VectorSubcoreMesh(
    core_axis_name="c", subcore_axis_name="s", num_cores=NC, num_subcores=NS
)


@functools.partial(
    pl.kernel,
    out_type=jax.ShapeDtypeStruct((T, D), jnp.float32),
    mesh=_sc_mesh,
    scratch_types=[
        pltpu.VMEM((K, D), jnp.float32),
        pltpu.VMEM((NCHUNK, CHUNK), jnp.int32),
        [pltpu.VMEM((CHUNK, D), jnp.float32)] * NBUF,
        [pltpu.SemaphoreType.DMA] * NBUF,
    ],
)
def _sc_build(p_hbm, tok_hbm, out_hbm, p_v, tok_v, rows, wsem):
    wid = lax.axis_index("s") * NC + lax.axis_index("c")
    pltpu.sync_copy(p_hbm, p_v)
    pltpu.sync_copy(tok_hbm.at[wid], tok_v)
    base = wid * TOK_PER_W

    def round_body(rd, carry):
        for bb in range(NBUF):
            j = rd * NBUF + bb

            @pl.when(rd > 0)
            def _():
                pltpu.make_async_copy(
                    rows[bb], out_hbm.at[pl.ds(base, CHUNK)], wsem[bb]
                ).wait()

            def grp(g, c2):
                tv = tok_v[j, pl.ds(g * 16, 16)]
                for r in range(16):
                    t = tv[r]
                    row = g * 16 + r
                    for dg in range(D // 16):
                        rows[bb][row, pl.ds(dg * 16, 16)] = p_v[t, pl.ds(dg * 16, 16)]
                return c2

            lax.fori_loop(0, CHUNK // 16, grp, 0)
            pltpu.async_copy(
                rows[bb], out_hbm.at[pl.ds(base + j * CHUNK, CHUNK)], wsem[bb]
            )
        return carry

    lax.fori_loop(0, NCHUNK // NBUF, round_body, 0)
    for bb in range(NBUF):
        pltpu.make_async_copy(
            rows[bb], out_hbm.at[pl.ds(base, CHUNK)], wsem[bb]
        ).wait()


def kernel(tokens, table, W, b):
    P = _project_table(table, W, b)
    tok3 = tokens.astype(jnp.int32).reshape(NW, NCHUNK, CHUNK)
    return _sc_build(P, tok3)
